# baseline (speedup 1.0000x reference)
import functools

import jax
import jax.numpy as jnp
from jax import lax
from jax.experimental import pallas as pl
from jax.experimental.pallas import tpu as pltpu
from jax.experimental.pallas import tpu_sc as plsc

F = 4
V = 100002
D = 64
NC = 2
NS = 16
NW = NC * NS
CH = 128


def _make_sc_kernel(n_tok):
    wpf = NW // F
    per_w = n_tok // wpf
    nch = per_w // CH
    mesh = plsc.VectorSubcoreMesh(core_axis_name="c", subcore_axis_name="s")

    @functools.partial(
        pl.kernel,
        out_type=[
            jax.ShapeDtypeStruct((F * n_tok, D), jnp.float32),
            jax.ShapeDtypeStruct((F * n_tok,), jnp.int32),
        ],
        mesh=mesh,
        compiler_params=pltpu.CompilerParams(use_tc_tiling_on_sc=False),
        scratch_types=[
            pltpu.VMEM((per_w,), jnp.int32),
            pltpu.VMEM((CH,), jnp.int32),
            pltpu.VMEM((CH,), jnp.int32),
            pltpu.VMEM((CH,), jnp.int32),
            pltpu.VMEM((CH, D), jnp.float32),
            pltpu.SemaphoreType.DMA,
        ],
    )
    def sc_kernel(tok_hbm, map_hbm, tab_hbm, out_hbm, oidx_hbm,
                  tok_v, midx_v, fidx_v, grow_v, rows_v, sem):
        c = lax.axis_index("c")
        s = lax.axis_index("s")
        wid = s * NC + c
        f = wid // wpf
        slot = wid - f * wpf
        tbase = slot * per_w
        obase = f * n_tok + tbase
        foff = f * V
        pltpu.sync_copy(tok_hbm.at[pl.ds(tbase, per_w)], tok_v)

        def chunk(j, carry):
            cb = j * CH
            for i in range(CH // 16):
                midx_v[pl.ds(i * 16, 16)] = tok_v[pl.ds(cb + i * 16, 16)] + foff
            pltpu.async_copy(map_hbm.at[midx_v], fidx_v, sem).wait()
            pltpu.sync_copy(fidx_v, oidx_hbm.at[pl.ds(obase + cb, CH)])
            for i in range(CH // 16):
                grow_v[pl.ds(i * 16, 16)] = fidx_v[pl.ds(i * 16, 16)] + foff
            pltpu.async_copy(tab_hbm.at[grow_v], rows_v, sem).wait()
            pltpu.sync_copy(rows_v, out_hbm.at[pl.ds(obase + cb, CH)])
            return carry

        lax.fori_loop(0, nch, chunk, 0)

    return sc_kernel


@jax.jit
def kernel(token_seqs, tables, mappings):
    b, s = token_seqs.shape
    n_tok = b * s
    tok_flat = token_seqs.reshape(n_tok)
    map_flat = mappings.reshape(F * V)
    tab_flat = tables.reshape(F * V, D)
    out_flat, oidx_flat = _make_sc_kernel(n_tok)(tok_flat, map_flat, tab_flat)
    out_tensor = out_flat.reshape(F, b, s, D)
    out_indices = oidx_flat.reshape(F, b, s)
    return (out_tensor, out_indices)


# speedup vs baseline: 1.0027x; 1.0027x over previous
import functools

import jax
import jax.numpy as jnp
from jax import lax
from jax.experimental import pallas as pl
from jax.experimental.pallas import tpu as pltpu
from jax.experimental.pallas import tpu_sc as plsc

F = 4
V = 100002
D = 64
NC = 2
NS = 16
NW = NC * NS
CH = 128


def _make_sc_kernel(n_tok):
    wpf = NW // F
    per_w = n_tok // wpf
    nch = per_w // CH
    mesh = plsc.VectorSubcoreMesh(core_axis_name="c", subcore_axis_name="s")

    @functools.partial(
        pl.kernel,
        out_type=[
            jax.ShapeDtypeStruct((F * n_tok, D), jnp.float32),
            jax.ShapeDtypeStruct((F * n_tok,), jnp.int32),
        ],
        mesh=mesh,
        compiler_params=pltpu.CompilerParams(use_tc_tiling_on_sc=False),
        scratch_types=[
            pltpu.VMEM((per_w,), jnp.int32),
            pltpu.VMEM((CH,), jnp.int32),
            pltpu.VMEM((CH, D), jnp.float32),
            pltpu.SemaphoreType.DMA,
        ],
    )
    def sc_kernel(tok_hbm, map_hbm, tab_hbm, out_hbm, oidx_hbm,
                  tok_v, fidx_v, rows_v, sem):
        c = lax.axis_index("c")
        s = lax.axis_index("s")
        wid = s * NC + c
        f = wid // wpf
        slot = wid - f * wpf
        tbase = slot * per_w
        obase = f * n_tok + tbase
        map_f = map_hbm.at[f]
        tab_f = tab_hbm.at[f]
        pltpu.sync_copy(tok_hbm.at[pl.ds(tbase, per_w)], tok_v)

        def chunk(j, carry):
            cb = j * CH
            pltpu.async_copy(map_f.at[tok_v.at[pl.ds(cb, CH)]], fidx_v, sem).wait()
            pltpu.sync_copy(fidx_v, oidx_hbm.at[pl.ds(obase + cb, CH)])
            pltpu.async_copy(tab_f.at[fidx_v], rows_v, sem).wait()
            pltpu.sync_copy(rows_v, out_hbm.at[pl.ds(obase + cb, CH)])
            return carry

        lax.fori_loop(0, nch, chunk, 0)

    return sc_kernel


@jax.jit
def kernel(token_seqs, tables, mappings):
    b, s = token_seqs.shape
    n_tok = b * s
    tok_flat = token_seqs.reshape(n_tok)
    out_flat, oidx_flat = _make_sc_kernel(n_tok)(tok_flat, mappings, tables)
    out_tensor = out_flat.reshape(F, b, s, D)
    out_indices = oidx_flat.reshape(F, b, s)
    return (out_tensor, out_indices)


# speedup vs baseline: 2.8022x; 2.7947x over previous
import functools

import jax
import jax.numpy as jnp
from jax import lax
from jax.experimental import pallas as pl
from jax.experimental.pallas import tpu as pltpu
from jax.experimental.pallas import tpu_sc as plsc

F = 4
V = 100002
D = 64
DP = 128
NC = 2
NS = 16
NW = NC * NS
CH = 128


def _make_sc_kernel(n_tok):
    wpf = NW // F
    per_w = n_tok // wpf
    nch = per_w // CH
    mesh = plsc.VectorSubcoreMesh(core_axis_name="c", subcore_axis_name="s")

    @functools.partial(
        pl.kernel,
        out_type=[
            jax.ShapeDtypeStruct((F * n_tok, DP), jnp.float32),
            jax.ShapeDtypeStruct((F * n_tok,), jnp.int32),
        ],
        mesh=mesh,
        scratch_types=[
            pltpu.VMEM((per_w,), jnp.int32),
            pltpu.VMEM((CH,), jnp.int32),
            pltpu.VMEM((CH,), jnp.int32),
            pltpu.VMEM((CH, DP), jnp.float32),
            pltpu.SemaphoreType.DMA,
        ],
    )
    def sc_kernel(tok_hbm, map_hbm, tab_hbm, out_hbm, oidx_hbm,
                  tok_v, midx_v, fidx_v, rows_v, sem):
        c = lax.axis_index("c")
        s = lax.axis_index("s")
        wid = s * NC + c
        f = wid // wpf
        slot = wid - f * wpf
        tbase = slot * per_w
        obase = f * n_tok + tbase
        foff = f * V
        tab_f = tab_hbm.at[f]
        pltpu.sync_copy(tok_hbm.at[pl.ds(tbase, per_w)], tok_v)

        def chunk(j, carry):
            cb = j * CH
            for i in range(CH // 16):
                midx_v[pl.ds(i * 16, 16)] = tok_v[pl.ds(cb + i * 16, 16)] + foff
            pltpu.async_copy(map_hbm.at[midx_v], fidx_v, sem).wait()
            pltpu.sync_copy(fidx_v, oidx_hbm.at[pl.ds(obase + cb, CH)])
            pltpu.async_copy(tab_f.at[fidx_v], rows_v, sem).wait()
            pltpu.sync_copy(rows_v, out_hbm.at[pl.ds(obase + cb, CH)])
            return carry

        lax.fori_loop(0, nch, chunk, 0)

    return sc_kernel


@jax.jit
def kernel(token_seqs, tables, mappings):
    b, s = token_seqs.shape
    n_tok = b * s
    tok_flat = token_seqs.reshape(n_tok)
    map_flat = mappings.reshape(F * V)
    tab128 = jnp.pad(tables, ((0, 0), (0, 0), (0, DP - D)))
    out_pad, oidx_flat = _make_sc_kernel(n_tok)(tok_flat, map_flat, tab128)
    out_tensor = out_pad[:, :D].reshape(F, b, s, D)
    out_indices = oidx_flat.reshape(F, b, s)
    return (out_tensor, out_indices)
